# baseline (speedup 1.0000x reference)
import jax
import jax.numpy as jnp
from jax import lax
from jax.experimental import pallas as pl
from jax.experimental.pallas import tpu as pltpu
from jax.experimental.pallas import tpu_sc as plsc

_R = 256
_C = 512
_DEPTH = 8


def _sc_body(emb_hbm, out_hbm, emb_v, strip_v, sem):
    Sq = out_hbm.shape[0]
    n_emb = emb_v.shape[0]
    max_pos = (n_emb - 1) // 2
    d = emb_v.shape[1]
    strip_rows = _C + _R - 1

    wid = lax.axis_index("c") * 16 + lax.axis_index("s")
    ib = wid // 4
    jb = wid % 4
    i0 = ib * _R
    j0 = jb * _C
    s_lo = (Sq - 1) - (i0 + _R - 1) + j0

    pltpu.sync_copy(emb_hbm, emb_v)

    def build(t, carry):
        u = jnp.clip(s_lo + t - (Sq - 1), -max_pos, max_pos) + max_pos
        for m in range(d // 16):
            strip_v[t, pl.ds(m * 16, 16)] = emb_v[u, pl.ds(m * 16, 16)]
        return carry

    lax.fori_loop(0, strip_rows, build, 0)

    def descr(r):
        return pltpu.make_async_copy(
            strip_v.at[pl.ds((_R - 1) - r, _C), :],
            out_hbm.at[i0 + r, pl.ds(j0, _C), :],
            sem,
        )

    def fire(r, carry):
        @pl.when(r >= _DEPTH)
        def _():
            descr(r - _DEPTH).wait()

        descr(r).start()
        return carry

    lax.fori_loop(0, _R, fire, 0)
    for r in range(_R - _DEPTH, _R):
        descr(r).wait()


def kernel(q, v, embeddings):
    Sq = q.shape[1]
    Sv = v.shape[1]
    n_emb, d = embeddings.shape
    mesh = plsc.VectorSubcoreMesh(core_axis_name="c", subcore_axis_name="s")
    run = pl.kernel(
        _sc_body,
        out_type=jax.ShapeDtypeStruct((Sq, Sv, d), embeddings.dtype),
        mesh=mesh,
        scratch_types=[
            pltpu.VMEM((n_emb, d), embeddings.dtype),
            pltpu.VMEM((_C + _R, d), embeddings.dtype),
            pltpu.SemaphoreType.DMA,
        ],
        compiler_params=pltpu.CompilerParams(use_tc_tiling_on_sc=True),
    )
    return run(embeddings)


# speedup vs baseline: 5.5582x; 5.5582x over previous
import jax
import jax.numpy as jnp
from jax.experimental import pallas as pl
from jax.experimental.pallas import tpu as pltpu

_R = 16


def _body(embt_ref, out_ref, bt_ref):
    Sq = pl.num_programs(0) * _R
    Sv = out_ref.shape[2]
    d, n_emb = embt_ref.shape
    max_pos = (n_emb - 1) // 2
    lo = Sq - max_pos
    hi = Sq + max_pos

    p = pl.program_id(0)

    @pl.when(p == 0)
    def _():
        e = embt_ref[...]
        bt_ref[:, 0:lo] = jnp.broadcast_to(e[:, 0:1], (d, lo))
        bt_ref[:, lo:hi] = e[:, 1:n_emb]
        bt_ref[:, hi:] = jnp.broadcast_to(
            e[:, n_emb - 1 : n_emb], (d, bt_ref.shape[1] - hi)
        )

    i0 = p * _R
    t = (Sq - _R) - i0
    rot = pltpu.roll(bt_ref[...], -t, axis=1)
    for r in range(_R):
        off = (_R - 1) - r
        out_ref[r, :, :] = rot[:, off : off + Sv]


def kernel(q, v, embeddings):
    Sq = q.shape[1]
    Sv = v.shape[1]
    n_emb, d = embeddings.shape
    out = pl.pallas_call(
        _body,
        grid=(Sq // _R,),
        in_specs=[pl.BlockSpec((d, n_emb), lambda p: (0, 0))],
        out_specs=pl.BlockSpec((_R, d, Sv), lambda p: (p, 0, 0)),
        out_shape=jax.ShapeDtypeStruct((Sq, d, Sv), embeddings.dtype),
        scratch_shapes=[pltpu.VMEM((d, Sq + Sv), embeddings.dtype)],
    )(embeddings.T)
    return out.transpose(0, 2, 1)


# speedup vs baseline: 6.0333x; 1.0855x over previous
import jax
import jax.numpy as jnp
from jax.experimental import pallas as pl
from jax.experimental.pallas import tpu as pltpu

_R = 32


def _body(embt_ref, out_ref, bt_ref):
    Sq = pl.num_programs(0) * _R
    Sv = out_ref.shape[2]
    d, n_emb = embt_ref.shape
    max_pos = (n_emb - 1) // 2
    lo = Sq - max_pos
    hi = Sq + max_pos

    p = pl.program_id(0)

    @pl.when(p == 0)
    def _():
        e = embt_ref[...]
        bt_ref[:, 0:lo] = jnp.broadcast_to(e[:, 0:1], (d, lo))
        bt_ref[:, lo:hi] = e[:, 1:n_emb]
        bt_ref[:, hi:] = jnp.broadcast_to(
            e[:, n_emb - 1 : n_emb], (d, bt_ref.shape[1] - hi)
        )

    i0 = p * _R
    t = (Sq - _R) - i0
    rot = pltpu.roll(bt_ref[...], -t, axis=1)
    for r in range(_R):
        off = (_R - 1) - r
        out_ref[r, :, :] = rot[:, off : off + Sv]


def kernel(q, v, embeddings):
    Sq = q.shape[1]
    Sv = v.shape[1]
    n_emb, d = embeddings.shape
    out = pl.pallas_call(
        _body,
        grid=(Sq // _R,),
        in_specs=[pl.BlockSpec((d, n_emb), lambda p: (0, 0))],
        out_specs=pl.BlockSpec((_R, d, Sv), lambda p: (p, 0, 0)),
        out_shape=jax.ShapeDtypeStruct((Sq, d, Sv), embeddings.dtype),
        scratch_shapes=[pltpu.VMEM((d, Sq + Sv), embeddings.dtype)],
    )(embeddings.T)
    return out.transpose(0, 2, 1)
